# Initial kernel scaffold; baseline (speedup 1.0000x reference)
import jax
import jax.numpy as jnp
from jax.experimental import pallas as pl


def kernel(x, edge_index, batch, W1, b1, W2, b2, M0, mb0, M1, mb1, M2, mb2, M3, mb3):
    raise NotImplementedError("write your pallas kernel here")


# speedup vs baseline: 35.4406x; 35.4406x over previous
import functools

import jax
import jax.numpy as jnp
from jax import lax
from jax.experimental import pallas as pl
from jax.experimental.pallas import tpu as pltpu
from jax.experimental.pallas import tpu_sc as plsc

N = 10000
NP = 10240
PAD = NP - N
E = 320000
NUM_GRAPHS = 64
NC = 2
NS = 16
NW = NC * NS
EPT = E // NW
CHUNK = 80
NCHUNK = EPT // CHUNK
ROWS_PT = NP // NS
DEGW = 16


def _mesh():
    return plsc.VectorSubcoreMesh(
        core_axis_name="c", subcore_axis_name="s",
        num_cores=NC, num_subcores=NS)


def _deg_body(dst3d_hbm, degp_hbm, dstb, ones_v, zbuf, deg_sp):
    c = lax.axis_index("c")
    s = lax.axis_index("s")
    wid = c * NS + s

    def fill(i, _):
        zbuf[i] = jnp.zeros((DEGW,), jnp.float32)
        return 0
    lax.fori_loop(0, ROWS_PT, fill, 0)

    def fill1(i, _):
        ones_v[i] = jnp.ones((DEGW,), jnp.float32)
        return 0
    lax.fori_loop(0, CHUNK, fill1, 0)

    pltpu.sync_copy(zbuf, deg_sp.at[pl.ds(s * ROWS_PT, ROWS_PT)])
    pltpu.sync_copy(dst3d_hbm.at[wid], dstb)
    plsc.subcore_barrier()

    def body(j, _):
        pltpu.sync_copy(ones_v, deg_sp.at[dstb.at[j]], add=True)
        return 0
    lax.fori_loop(0, NCHUNK, body, 0)

    plsc.subcore_barrier()
    pltpu.sync_copy(deg_sp.at[pl.ds(s * ROWS_PT, ROWS_PT)],
                    degp_hbm.at[c, pl.ds(s * ROWS_PT, ROWS_PT)])


def _deg(dst3d):
    return pl.kernel(
        _deg_body,
        out_type=jax.ShapeDtypeStruct((NC, NP, DEGW), jnp.float32),
        mesh=_mesh(),
        compiler_params=pltpu.CompilerParams(use_tc_tiling_on_sc=False),
        scratch_types=[
            pltpu.VMEM((NCHUNK, CHUNK), jnp.int32),
            pltpu.VMEM((CHUNK, DEGW), jnp.float32),
            pltpu.VMEM((ROWS_PT, DEGW), jnp.float32),
            pltpu.VMEM_SHARED((NP, DEGW), jnp.float32),
        ],
    )(dst3d)


def _prop_body(hs_hbm, src3d_hbm, dst3d_hbm, part_hbm,
               srcb, dstb, rows0, rows1, acc_sp, sem0, sem1):
    c = lax.axis_index("c")
    s = lax.axis_index("s")
    wid = c * NS + s

    pltpu.sync_copy(hs_hbm.at[pl.ds(s * ROWS_PT, ROWS_PT)],
                    acc_sp.at[pl.ds(s * ROWS_PT, ROWS_PT)])
    pltpu.sync_copy(src3d_hbm.at[wid], srcb)
    pltpu.sync_copy(dst3d_hbm.at[wid], dstb)
    pltpu.async_copy(hs_hbm.at[srcb.at[0]], rows0, sem0)
    plsc.subcore_barrier()

    npair = (NCHUNK - 1) // 2

    def body(jj, _):
        j0 = 2 * jj
        j1 = j0 + 1
        pltpu.async_copy(hs_hbm.at[srcb.at[j1]], rows1, sem1)
        pltpu.make_async_copy(hs_hbm.at[srcb.at[j0]], rows0, sem0).wait()
        pltpu.sync_copy(rows0, acc_sp.at[dstb.at[j0]], add=True)
        pltpu.async_copy(hs_hbm.at[srcb.at[j0 + 2]], rows0, sem0)
        pltpu.make_async_copy(hs_hbm.at[srcb.at[j1]], rows1, sem1).wait()
        pltpu.sync_copy(rows1, acc_sp.at[dstb.at[j1]], add=True)
        return 0
    lax.fori_loop(0, npair, body, 0)

    last = NCHUNK - 1
    pltpu.make_async_copy(hs_hbm.at[srcb.at[last]], rows0, sem0).wait()
    pltpu.sync_copy(rows0, acc_sp.at[dstb.at[last]], add=True)

    plsc.subcore_barrier()
    pltpu.sync_copy(acc_sp.at[pl.ds(s * ROWS_PT, ROWS_PT)],
                    part_hbm.at[c, pl.ds(s * ROWS_PT, ROWS_PT)])


def _prop(hs, src3d, dst3d):
    return pl.kernel(
        _prop_body,
        out_type=jax.ShapeDtypeStruct((NC, NP, 64), jnp.float32),
        mesh=_mesh(),
        compiler_params=pltpu.CompilerParams(use_tc_tiling_on_sc=False),
        scratch_types=[
            pltpu.VMEM((NCHUNK, CHUNK), jnp.int32),
            pltpu.VMEM((NCHUNK, CHUNK), jnp.int32),
            pltpu.VMEM((CHUNK, 64), jnp.float32),
            pltpu.VMEM((CHUNK, 64), jnp.float32),
            pltpu.VMEM_SHARED((NP, 64), jnp.float32),
            pltpu.SemaphoreType.DMA,
            pltpu.SemaphoreType.DMA,
        ],
    )(hs, src3d, dst3d)


def _dinv_from(degp):
    deg = degp[0] + degp[1]
    return lax.rsqrt(deg[:, :1] + 1.0)


def _mm_scale_body(x_ref, w_ref, degp_ref, hs_ref):
    dinv = _dinv_from(degp_ref[...])
    xw = jnp.dot(x_ref[...], w_ref[...], preferred_element_type=jnp.float32)
    hs_ref[pl.ds(0, N), :] = xw * dinv[:N]
    hs_ref[pl.ds(N, PAD), :] = jnp.zeros((PAD, 64), jnp.float32)


def _combine_body(a_ref, hs_ref, degp_ref, b1_ref, out_ref):
    dinv = _dinv_from(degp_ref[...])
    a = a_ref[...]
    hs = hs_ref[...]
    p = (a[0] + a[1] - hs) * dinv + b1_ref[...]
    out_ref[...] = jnp.maximum(p, 0.0) * dinv
    out_ref[pl.ds(N, PAD), :] = jnp.zeros((PAD, 64), jnp.float32)


def _leaky(v, alpha):
    return jnp.where(v > 0, v, alpha * v)


def _head_body(a_ref, hs_ref, degp_ref, w2_ref, b2_ref, batch_ref,
               m0_ref, mb0_ref, m1_ref, mb1_ref, m2_ref, mb2_ref,
               m3_ref, mb3_ref, out_ref):
    dinv = _dinv_from(degp_ref[...])
    a = a_ref[...]
    p2 = (a[0] + a[1] - hs_ref[...]) * dinv
    h2 = jnp.dot(p2, w2_ref[...], preferred_element_type=jnp.float32)
    h2 = jnp.maximum(h2 + b2_ref[...], 0.0)
    gids = lax.broadcasted_iota(jnp.int32, (NUM_GRAPHS, NP), 0)
    onehot = (batch_ref[...] == gids).astype(jnp.float32)
    sums = jnp.dot(onehot, h2, preferred_element_type=jnp.float32)
    cnt = jnp.sum(onehot, axis=1, keepdims=True)
    g = sums / jnp.maximum(cnt, 1.0)
    g = _leaky(jnp.dot(g, m0_ref[...], preferred_element_type=jnp.float32)
               + mb0_ref[...], 0.2)
    g = _leaky(jnp.dot(g, m1_ref[...], preferred_element_type=jnp.float32)
               + mb1_ref[...], 0.1)
    g = _leaky(jnp.dot(g, m2_ref[...], preferred_element_type=jnp.float32)
               + mb2_ref[...], 0.1)
    g = jnp.dot(g, m3_ref[...], preferred_element_type=jnp.float32)
    out_ref[...] = jnp.maximum(g + mb3_ref[...], 0.0)


def _tc_call(body, out_shape, *args):
    return pl.pallas_call(
        body,
        out_shape=jax.ShapeDtypeStruct(out_shape, jnp.float32),
    )(*args)


def kernel(x, edge_index, batch, W1, b1, W2, b2,
           M0, mb0, M1, mb1, M2, mb2, M3, mb3):
    src3d = edge_index[0].reshape(NW, NCHUNK, CHUNK)
    dst3d = edge_index[1].reshape(NW, NCHUNK, CHUNK)
    batch_pad = jnp.concatenate(
        [batch, jnp.full((PAD,), -1, jnp.int32)]).reshape(1, NP)

    degp = _deg(dst3d)
    hs1 = _tc_call(_mm_scale_body, (NP, 64),
                   x, W1, degp)
    part1 = _prop(hs1, src3d, dst3d)
    hs2 = _tc_call(_combine_body, (NP, 64),
                   part1, hs1, degp, b1.reshape(1, 64))
    part2 = _prop(hs2, src3d, dst3d)
    out = _tc_call(_head_body, (NUM_GRAPHS, 1),
                   part2, hs2, degp, W2, b2.reshape(1, 128), batch_pad,
                   M0, mb0.reshape(1, 64), M1, mb1.reshape(1, 64),
                   M2, mb2.reshape(1, 64), M3, mb3.reshape(1, 1))
    return out
